# baseline (speedup 1.0000x reference)
import functools

import jax
import jax.numpy as jnp
from jax import lax
from jax.experimental import pallas as pl
from jax.experimental.pallas import tpu as pltpu
from jax.experimental.pallas import tpu_sc as plsc

R = 128
V = 100000
L = 16
NW = 32
RPW = R // NW
NVEC = V // L
CH = 125
NCH = NVEC // CH
CHE = CH * L
UNR = 5
NIT = NVEC // UNR


def _tec_body(logits_hbm, params_hbm, out_hbm, row_v, csum_v, par_v, out_v):
    wid = lax.axis_index("s") * 2 + lax.axis_index("c")
    pltpu.sync_copy(params_hbm.at[wid], par_v)
    iota = lax.iota(jnp.int32, L)
    zeros_f = jnp.zeros((L,), jnp.float32)
    zeros_i = jnp.zeros((L,), jnp.int32)
    neg_inf = jnp.full((L,), -jnp.inf, jnp.float32)
    pos_inf = jnp.full((L,), jnp.inf, jnp.float32)
    pvec = par_v[...]

    def _lane(vec, idx):
        return jnp.sum(jnp.where(iota == idx, vec, jnp.zeros((L,), vec.dtype)))

    scale_s = _lane(pvec, RPW)
    is_zero = _lane(pvec, RPW + 1) > jnp.float32(0.5)
    is_unit = _lane(pvec, RPW + 2) > jnp.float32(0.5)
    a16 = jnp.broadcast_to(scale_s, (L,))

    results = zeros_i
    for kr in range(RPW):
        row = wid * RPW + kr
        pltpu.sync_copy(logits_hbm.at[row], row_v)
        u_s = _lane(pvec, kr)

        def _greedy_path():
            def p1(i, c):
                gm, gi = c
                for uu in range(UNR):
                    v = row_v[pl.ds((i * UNR + uu) * L, L)]
                    upd = v > gm
                    gm = jnp.where(upd, v, gm)
                    gi = jnp.where(upd, iota + (i * UNR + uu) * L, gi)
                return gm, gi

            gm, gi = lax.fori_loop(0, NIT, p1, (neg_inf, zeros_i))
            g16 = jnp.broadcast_to(jnp.max(gm), (L,))
            big = jnp.full((L,), jnp.int32(V), jnp.int32)
            return jnp.min(jnp.where(gm == g16, gi, big))

        def _sample_path():
            def p1(i, c):
                mx = list(c[:UNR])
                mn = list(c[UNR:])
                for uu in range(UNR):
                    v = row_v[pl.ds((i * UNR + uu) * L, L)]
                    mx[uu] = jnp.maximum(mx[uu], v)
                    mn[uu] = jnp.minimum(mn[uu], v)
                return tuple(mx) + tuple(mn)

            accs = lax.fori_loop(
                0, NIT, p1, (neg_inf,) * UNR + (pos_inf,) * UNR
            )
            mx, mn = accs[0], accs[UNR]
            for q in range(1, UNR):
                mx = jnp.maximum(mx, accs[q])
                mn = jnp.minimum(mn, accs[UNR + q])
            m_hi = jnp.max(mx) * scale_s
            m_lo = jnp.min(mn) * scale_s
            m_s = jnp.maximum(m_hi, m_lo)
            b16 = jnp.broadcast_to(-m_s, (L,))

            def _p2(use_scale):
                def p2(c, carry):
                    def inner(j, s):
                        s = list(s)
                        for uu in range(UNR):
                            v = row_v[pl.ds((c * CH + j * UNR + uu) * L, L)]
                            if use_scale:
                                s[uu] = s[uu] + jnp.exp(v * a16 + b16)
                            else:
                                s[uu] = s[uu] + jnp.exp(v + b16)
                        return tuple(s)

                    s = lax.fori_loop(0, CH // UNR, inner, (zeros_f,) * UNR)
                    s16 = (s[0] + s[1]) + (s[2] + s[3]) + s[4]
                    csum_v[pl.ds(c * L, L)] = s16
                    return carry + s16

                return lax.fori_loop(0, NCH, p2, zeros_f)

            z16 = lax.cond(is_unit, lambda: _p2(False), lambda: _p2(True))
            z_s = jnp.sum(z16)
            t_s = u_s * z_s
            t16 = jnp.broadcast_to(t_s, (L,))

            def p3a(c, carry):
                acc, nfull, before = carry
                cs = jnp.sum(csum_v[pl.ds(c * L, L)])
                acc2 = acc + cs
                below = acc2 < t_s
                nfull = nfull + jnp.where(below, jnp.int32(1), jnp.int32(0))
                before = before + jnp.where(below, cs, jnp.float32(0.0))
                return acc2, nfull, before

            _, nfull, before = lax.fori_loop(
                0, NCH, p3a, (jnp.float32(0.0), jnp.int32(0), jnp.float32(0.0))
            )
            k = jnp.minimum(nfull, jnp.int32(NCH - 1))

            def p3b(j, carry):
                acc, cnt = carry
                v = row_v[pl.ds((k * CH + j) * L, L)]
                e = jnp.exp(v * a16 + b16)
                pre = plsc.cumsum(e) + jnp.broadcast_to(acc, (L,))
                cnt = cnt + plsc.all_reduce_population_count(pre < t16)
                return acc + jnp.sum(e), cnt

            _, cnt16 = lax.fori_loop(0, CH, p3b, (before, zeros_i))
            detail = jnp.max(cnt16)
            return jnp.where(
                nfull >= jnp.int32(NCH), jnp.int32(V), k * jnp.int32(CHE) + detail
            )

        res = lax.cond(is_zero, _greedy_path, _sample_path)
        results = jnp.where(iota == kr, jnp.broadcast_to(res, (L,)), results)

    out_v[...] = results
    pltpu.sync_copy(out_v, out_hbm.at[wid])


_sc_sample = functools.partial(
    pl.kernel,
    out_type=jax.ShapeDtypeStruct((NW, L), jnp.int32),
    mesh=plsc.VectorSubcoreMesh(core_axis_name="c", subcore_axis_name="s"),
    compiler_params=pltpu.CompilerParams(needs_layout_passes=False),
    scratch_types=[
        pltpu.VMEM((V,), jnp.float32),
        pltpu.VMEM((NCH * L,), jnp.float32),
        pltpu.VMEM((L,), jnp.float32),
        pltpu.VMEM((L,), jnp.int32),
    ],
)(_tec_body)


def kernel(logits, temperature):
    t = jnp.asarray(temperature).astype(jnp.float32)
    is_zero = t == jnp.float32(0.0)
    safe = jnp.where(is_zero, jnp.float32(1.0), t)
    scale = jnp.float32(1.0) / safe
    is_unit = scale == jnp.float32(1.0)
    u = jax.random.uniform(jax.random.key(42), (R, 1), dtype=jnp.float32)[:, 0]
    params = jnp.zeros((NW, L), jnp.float32)
    params = params.at[:, 0:RPW].set(u.reshape(NW, RPW))
    params = params.at[:, RPW].set(scale)
    params = params.at[:, RPW + 1].set(is_zero.astype(jnp.float32))
    params = params.at[:, RPW + 2].set(is_unit.astype(jnp.float32))
    out2d = _sc_sample(logits, params)
    return out2d[:, :RPW].reshape(R)


# speedup vs baseline: 1.0229x; 1.0229x over previous
import functools

import jax
import jax.numpy as jnp
from jax import lax
from jax.experimental import pallas as pl
from jax.experimental.pallas import tpu as pltpu
from jax.experimental.pallas import tpu_sc as plsc

R = 128
V = 100000
L = 16
NW = 32
RPW = R // NW
NVEC = V // L
CH = 125
NCH = NVEC // CH
CHE = CH * L
UNR = 5
NIT = NVEC // UNR


def _tec_body(logits_hbm, params_hbm, out_hbm, row_v, csum_v, par_v, out_v):
    wid = lax.axis_index("s") * 2 + lax.axis_index("c")
    pltpu.sync_copy(params_hbm.at[wid], par_v)
    iota = lax.iota(jnp.int32, L)
    zeros_f = jnp.zeros((L,), jnp.float32)
    zeros_i = jnp.zeros((L,), jnp.int32)
    neg_inf = jnp.full((L,), -jnp.inf, jnp.float32)
    pos_inf = jnp.full((L,), jnp.inf, jnp.float32)
    pvec = par_v[...]

    def _lane(vec, idx):
        return jnp.sum(jnp.where(iota == idx, vec, jnp.zeros((L,), vec.dtype)))

    scale_s = _lane(pvec, RPW)
    is_zero = _lane(pvec, RPW + 1) > jnp.float32(0.5)
    is_unit = _lane(pvec, RPW + 2) > jnp.float32(0.5)
    a16 = jnp.broadcast_to(scale_s, (L,))

    results = zeros_i
    for kr in range(RPW):
        row = wid * RPW + kr
        pltpu.sync_copy(logits_hbm.at[row], row_v)
        u_s = _lane(pvec, kr)

        def _greedy_path():
            def p1(i, c):
                gm, gi = c
                for uu in range(UNR):
                    v = row_v[pl.ds((i * UNR + uu) * L, L)]
                    upd = v > gm
                    gm = jnp.where(upd, v, gm)
                    gi = jnp.where(upd, iota + (i * UNR + uu) * L, gi)
                return gm, gi

            gm, gi = lax.fori_loop(0, NIT, p1, (neg_inf, zeros_i))
            g16 = jnp.broadcast_to(jnp.max(gm), (L,))
            big = jnp.full((L,), jnp.int32(V), jnp.int32)
            return jnp.min(jnp.where(gm == g16, gi, big))

        def _sample_path():
            U1 = 10
            def p1(i, c):
                mx = list(c[:UNR])
                mn = list(c[UNR:])
                for uu in range(U1):
                    v = row_v[pl.ds((i * U1 + uu) * L, L)]
                    mx[uu % UNR] = jnp.maximum(mx[uu % UNR], v)
                    mn[uu % UNR] = jnp.minimum(mn[uu % UNR], v)
                return tuple(mx) + tuple(mn)

            accs = lax.fori_loop(
                0, NVEC // U1, p1, (neg_inf,) * UNR + (pos_inf,) * UNR
            )
            mx, mn = accs[0], accs[UNR]
            for q in range(1, UNR):
                mx = jnp.maximum(mx, accs[q])
                mn = jnp.minimum(mn, accs[UNR + q])
            m_hi = jnp.max(mx) * scale_s
            m_lo = jnp.min(mn) * scale_s
            m_s = jnp.maximum(m_hi, m_lo)
            b16 = jnp.broadcast_to(-m_s, (L,))

            def _p2(use_scale):
                def p2(c, carry):
                    def inner(j, s):
                        s = list(s)
                        for uu in range(UNR):
                            v = row_v[pl.ds((c * CH + j * UNR + uu) * L, L)]
                            if use_scale:
                                s[uu] = s[uu] + jnp.exp(v * a16 + b16)
                            else:
                                s[uu] = s[uu] + jnp.exp(v + b16)
                        return tuple(s)

                    s = lax.fori_loop(0, CH // UNR, inner, (zeros_f,) * UNR)
                    s16 = (s[0] + s[1]) + (s[2] + s[3]) + s[4]
                    csum_v[pl.ds(c * L, L)] = s16
                    return carry + s16

                return lax.fori_loop(0, NCH, p2, zeros_f)

            z16 = lax.cond(is_unit, lambda: _p2(False), lambda: _p2(True))
            z_s = jnp.sum(z16)
            t_s = u_s * z_s
            t16 = jnp.broadcast_to(t_s, (L,))

            def p3a(c, carry):
                acc, nfull, before = carry
                cs = jnp.sum(csum_v[pl.ds(c * L, L)])
                acc2 = acc + cs
                below = acc2 < t_s
                nfull = nfull + jnp.where(below, jnp.int32(1), jnp.int32(0))
                before = before + jnp.where(below, cs, jnp.float32(0.0))
                return acc2, nfull, before

            _, nfull, before = lax.fori_loop(
                0, NCH, p3a, (jnp.float32(0.0), jnp.int32(0), jnp.float32(0.0))
            )
            k = jnp.minimum(nfull, jnp.int32(NCH - 1))

            def p3b(j, carry):
                acc, cnt = carry
                v = row_v[pl.ds((k * CH + j) * L, L)]
                e = jnp.exp(v * a16 + b16)
                pre = plsc.cumsum(e) + jnp.broadcast_to(acc, (L,))
                cnt = cnt + plsc.all_reduce_population_count(pre < t16)
                return acc + jnp.sum(e), cnt

            _, cnt16 = lax.fori_loop(0, CH, p3b, (before, zeros_i))
            detail = jnp.max(cnt16)
            return jnp.where(
                nfull >= jnp.int32(NCH), jnp.int32(V), k * jnp.int32(CHE) + detail
            )

        res = lax.cond(is_zero, _greedy_path, _sample_path)
        results = jnp.where(iota == kr, jnp.broadcast_to(res, (L,)), results)

    out_v[...] = results
    pltpu.sync_copy(out_v, out_hbm.at[wid])


_sc_sample = functools.partial(
    pl.kernel,
    out_type=jax.ShapeDtypeStruct((NW, L), jnp.int32),
    mesh=plsc.VectorSubcoreMesh(core_axis_name="c", subcore_axis_name="s"),
    compiler_params=pltpu.CompilerParams(needs_layout_passes=False),
    scratch_types=[
        pltpu.VMEM((V,), jnp.float32),
        pltpu.VMEM((NCH * L,), jnp.float32),
        pltpu.VMEM((L,), jnp.float32),
        pltpu.VMEM((L,), jnp.int32),
    ],
)(_tec_body)


def kernel(logits, temperature):
    t = jnp.asarray(temperature).astype(jnp.float32)
    is_zero = t == jnp.float32(0.0)
    safe = jnp.where(is_zero, jnp.float32(1.0), t)
    scale = jnp.float32(1.0) / safe
    is_unit = scale == jnp.float32(1.0)
    u = jax.random.uniform(jax.random.key(42), (R, 1), dtype=jnp.float32)[:, 0]
    params = jnp.zeros((NW, L), jnp.float32)
    params = params.at[:, 0:RPW].set(u.reshape(NW, RPW))
    params = params.at[:, RPW].set(scale)
    params = params.at[:, RPW + 1].set(is_zero.astype(jnp.float32))
    params = params.at[:, RPW + 2].set(is_unit.astype(jnp.float32))
    out2d = _sc_sample(logits, params)
    return out2d[:, :RPW].reshape(R)
